# Initial kernel scaffold; baseline (speedup 1.0000x reference)
import jax
import jax.numpy as jnp
from jax.experimental import pallas as pl


def kernel(x, table, W, b):
    raise NotImplementedError("write your pallas kernel here")


# speedup vs baseline: 1.5743x; 1.5743x over previous
import functools

import jax
import jax.numpy as jnp
from jax import lax
from jax.experimental import pallas as pl
from jax.experimental.pallas import tpu as pltpu
from jax.experimental.pallas import tpu_sc as plsc

_NC = 2
_NS = 16
_LANES = 16
_BB = 8


def _sc_segment_sum(table, x):
    B, L = x.shape
    D = table.shape[1]
    nw = _NC * _NS
    n_per = B // nw
    n_blocks = n_per // _BB
    n_chunks = D // _LANES
    mesh = plsc.VectorSubcoreMesh(core_axis_name="c", subcore_axis_name="s")

    @functools.partial(
        pl.kernel,
        out_type=jax.ShapeDtypeStruct((B, D), jnp.float32),
        mesh=mesh,
        compiler_params=pltpu.CompilerParams(use_tc_tiling_on_sc=False),
        scratch_types=[
            pltpu.VMEM((_BB, L), jnp.int32),
            pltpu.VMEM((_BB, L, D), jnp.float32),
            pltpu.VMEM((_BB, D), jnp.float32),
            pltpu.SemaphoreType.DMA,
        ],
    )
    def k(table_hbm, x_hbm, out_hbm, idx_v, rows_v, acc_v, sem):
        wid = lax.axis_index("s") * _NC + lax.axis_index("c")
        base = wid * n_per

        @pl.loop(0, n_blocks)
        def _(blk):
            row0 = base + blk * _BB
            pltpu.sync_copy(x_hbm.at[pl.ds(row0, _BB)], idx_v)
            copies = [
                pltpu.async_copy(table_hbm.at[idx_v.at[i]], rows_v.at[i], sem)
                for i in range(_BB)
            ]
            for cp in copies:
                cp.wait()
            for i in range(_BB):
                accs = [rows_v[i, 0, pl.ds(c * _LANES, _LANES)]
                        for c in range(n_chunks)]
                for r in range(1, L):
                    for c in range(n_chunks):
                        accs[c] = accs[c] + rows_v[i, r, pl.ds(c * _LANES, _LANES)]
                for c in range(n_chunks):
                    acc_v[i, pl.ds(c * _LANES, _LANES)] = accs[c]
            pltpu.sync_copy(acc_v, out_hbm.at[pl.ds(row0, _BB)])

    return k(table, x)


def _tc_finish(x, rawsum, table, W, b):
    B, L = x.shape
    D = table.shape[1]

    def body(x_ref, s_ref, t0_ref, w_ref, b_ref, o_ref):
        zc = jnp.sum((x_ref[...] == 0).astype(jnp.float32), axis=1,
                     keepdims=True)
        pooled = (s_ref[...] - zc * t0_ref[0:1, :]) * (1.0 / L)
        o_ref[...] = lax.dot_general(
            pooled, w_ref[...], (((1,), (1,)), ((), ())),
            preferred_element_type=jnp.float32) + b_ref[...]

    tb = 2048
    return pl.pallas_call(
        body,
        out_shape=jax.ShapeDtypeStruct((B, L), jnp.float32),
        grid=(B // tb,),
        in_specs=[
            pl.BlockSpec((tb, L), lambda i: (i, 0)),
            pl.BlockSpec((tb, D), lambda i: (i, 0)),
            pl.BlockSpec((8, D), lambda i: (0, 0)),
            pl.BlockSpec((L, D), lambda i: (0, 0)),
            pl.BlockSpec((1, L), lambda i: (0, 0)),
        ],
        out_specs=pl.BlockSpec((tb, L), lambda i: (i, 0)),
    )(x, rawsum, table, W, b.reshape(1, L))


def kernel(x, table, W, b):
    x = x.astype(jnp.int32)
    rawsum = _sc_segment_sum(table, x)
    return _tc_finish(x, rawsum, table, W, b)


# speedup vs baseline: 1.7482x; 1.1105x over previous
import functools

import jax
import jax.numpy as jnp
from jax import lax
from jax.experimental import pallas as pl
from jax.experimental.pallas import tpu as pltpu
from jax.experimental.pallas import tpu_sc as plsc

_NC = 2
_NS = 16
_LANES = 16
_BB = 8


def _sc_segment_sum(table, x):
    B, L = x.shape
    D = table.shape[1]
    nw = _NC * _NS
    n_per = B // nw
    n_blocks = n_per // _BB
    n_chunks = D // _LANES
    mesh = plsc.VectorSubcoreMesh(core_axis_name="c", subcore_axis_name="s")

    @functools.partial(
        pl.kernel,
        out_type=jax.ShapeDtypeStruct((B, D), jnp.float32),
        mesh=mesh,
        compiler_params=pltpu.CompilerParams(use_tc_tiling_on_sc=False),
        scratch_types=[
            pltpu.VMEM((n_blocks, _BB * L), jnp.int32),
            pltpu.VMEM((2, _BB * L, D), jnp.float32),
            pltpu.VMEM((2, _BB, D), jnp.float32),
            pltpu.SemaphoreType.DMA,
            pltpu.SemaphoreType.DMA,
        ],
    )
    def k(table_hbm, x_hbm, out_hbm, idx_v, rows_v, acc_v, sem0, sem1):
        wid = lax.axis_index("s") * _NC + lax.axis_index("c")
        base = wid * n_per
        sems = (sem0, sem1)

        pltpu.sync_copy(x_hbm.at[pl.ds(wid * n_blocks, n_blocks)], idx_v)

        def gather(slot, blk):
            return pltpu.make_async_copy(
                table_hbm.at[idx_v.at[blk]],
                rows_v.at[slot], sems[slot])

        def compute(slot, blk):
            for i in range(_BB):
                accs = [rows_v[slot, i * L, pl.ds(c * _LANES, _LANES)]
                        for c in range(n_chunks)]
                accs2 = [rows_v[slot, i * L + 1, pl.ds(c * _LANES, _LANES)]
                         for c in range(n_chunks)]
                for r in range(2, L, 2):
                    for c in range(n_chunks):
                        accs[c] = accs[c] + rows_v[slot, i * L + r,
                                                   pl.ds(c * _LANES, _LANES)]
                for r in range(3, L, 2):
                    for c in range(n_chunks):
                        accs2[c] = accs2[c] + rows_v[slot, i * L + r,
                                                     pl.ds(c * _LANES, _LANES)]
                for c in range(n_chunks):
                    acc_v[slot, i, pl.ds(c * _LANES, _LANES)] = \
                        accs[c] + accs2[c]
            pltpu.sync_copy(acc_v.at[slot],
                            out_hbm.at[pl.ds(base + blk * _BB, _BB)])

        gather(0, 0).start()

        @pl.loop(0, n_blocks // 2)
        def _(j):
            blk = j * 2
            gather(1, blk + 1).start()
            gather(0, blk).wait()
            compute(0, blk)

            @pl.when(blk + 2 < n_blocks)
            def _():
                gather(0, blk + 2).start()

            gather(1, blk + 1).wait()
            compute(1, blk + 1)

    return k(table, x.reshape(B // _BB, _BB * L))


def _tc_finish(x, rawsum, table, W, b):
    B, L = x.shape
    D = table.shape[1]

    def body(x_ref, s_ref, t0_ref, w_ref, b_ref, o_ref):
        zc = jnp.sum((x_ref[...] == 0).astype(jnp.float32), axis=1,
                     keepdims=True)
        pooled = (s_ref[...] - zc * t0_ref[0:1, :]) * (1.0 / L)
        o_ref[...] = lax.dot_general(
            pooled, w_ref[...], (((1,), (1,)), ((), ())),
            preferred_element_type=jnp.float32) + b_ref[...]

    tb = 2048
    return pl.pallas_call(
        body,
        out_shape=jax.ShapeDtypeStruct((B, L), jnp.float32),
        grid=(B // tb,),
        in_specs=[
            pl.BlockSpec((tb, L), lambda i: (i, 0)),
            pl.BlockSpec((tb, D), lambda i: (i, 0)),
            pl.BlockSpec((8, D), lambda i: (0, 0)),
            pl.BlockSpec((L, D), lambda i: (0, 0)),
            pl.BlockSpec((1, L), lambda i: (0, 0)),
        ],
        out_specs=pl.BlockSpec((tb, L), lambda i: (i, 0)),
    )(x, rawsum, table, W, b.reshape(1, L))


def kernel(x, table, W, b):
    x = x.astype(jnp.int32)
    rawsum = _sc_segment_sum(table, x)
    return _tc_finish(x, rawsum, table, W, b)


# speedup vs baseline: 2.2874x; 1.3084x over previous
import functools

import jax
import jax.numpy as jnp
from jax import lax
from jax.experimental import pallas as pl
from jax.experimental.pallas import tpu as pltpu
from jax.experimental.pallas import tpu_sc as plsc

_NC = 2
_NS = 16
_LANES = 16
_BB = 8


def _sc_segment_sum(table, x):
    B, L = x.shape
    D = table.shape[1]
    nw = _NC * _NS
    n_per = B // nw
    n_blocks = n_per // _BB
    n_chunks = D // _LANES
    mesh = plsc.VectorSubcoreMesh(core_axis_name="c", subcore_axis_name="s")

    @functools.partial(
        pl.kernel,
        out_type=jax.ShapeDtypeStruct((B, D), jnp.float32),
        mesh=mesh,
        compiler_params=pltpu.CompilerParams(use_tc_tiling_on_sc=False),
        scratch_types=[
            pltpu.VMEM((n_blocks, _BB * L), jnp.int32),
            pltpu.VMEM((2, _BB * L, D), jnp.float32),
            pltpu.VMEM((2, _BB, D), jnp.float32),
            pltpu.SemaphoreType.DMA,
            pltpu.SemaphoreType.DMA,
        ],
    )
    def k(table_hbm, x_hbm, out_hbm, idx_v, rows_v, acc_v, sem0, sem1):
        wid = lax.axis_index("s") * _NC + lax.axis_index("c")
        base = wid * n_per
        sems = (sem0, sem1)

        pltpu.sync_copy(x_hbm.at[pl.ds(wid * n_blocks, n_blocks)], idx_v)

        def gather(slot, blk):
            return pltpu.make_async_copy(
                table_hbm.at[idx_v.at[blk]],
                rows_v.at[slot], sems[slot])

        def compute(slot, blk):
            @pl.loop(0, _BB)
            def _(i):
                init = tuple(rows_v[slot, i * L, pl.ds(c * _LANES, _LANES)]
                             for c in range(n_chunks))

                @plsc.parallel_loop(1, L, unroll=7, carry=init)
                def accs(r, carry):
                    return tuple(
                        carry[c] + rows_v[slot, i * L + r,
                                          pl.ds(c * _LANES, _LANES)]
                        for c in range(n_chunks))

                for c in range(n_chunks):
                    acc_v[slot, i, pl.ds(c * _LANES, _LANES)] = accs[c]
            pltpu.sync_copy(acc_v.at[slot],
                            out_hbm.at[pl.ds(base + blk * _BB, _BB)])

        gather(0, 0).start()

        @pl.loop(0, n_blocks // 2)
        def _(j):
            blk = j * 2
            gather(1, blk + 1).start()
            gather(0, blk).wait()
            compute(0, blk)

            @pl.when(blk + 2 < n_blocks)
            def _():
                gather(0, blk + 2).start()

            gather(1, blk + 1).wait()
            compute(1, blk + 1)

    return k(table, x.reshape(B // _BB, _BB * L))


def _tc_finish(x, rawsum, table, W, b):
    B, L = x.shape
    D = table.shape[1]

    def body(x_ref, s_ref, t0_ref, w_ref, b_ref, o_ref):
        zc = jnp.sum((x_ref[...] == 0).astype(jnp.float32), axis=1,
                     keepdims=True)
        pooled = (s_ref[...] - zc * t0_ref[0:1, :]) * (1.0 / L)
        o_ref[...] = lax.dot_general(
            pooled, w_ref[...], (((1,), (1,)), ((), ())),
            preferred_element_type=jnp.float32) + b_ref[...]

    tb = 2048
    return pl.pallas_call(
        body,
        out_shape=jax.ShapeDtypeStruct((B, L), jnp.float32),
        grid=(B // tb,),
        in_specs=[
            pl.BlockSpec((tb, L), lambda i: (i, 0)),
            pl.BlockSpec((tb, D), lambda i: (i, 0)),
            pl.BlockSpec((8, D), lambda i: (0, 0)),
            pl.BlockSpec((L, D), lambda i: (0, 0)),
            pl.BlockSpec((1, L), lambda i: (0, 0)),
        ],
        out_specs=pl.BlockSpec((tb, L), lambda i: (i, 0)),
    )(x, rawsum, table, W, b.reshape(1, L))


def kernel(x, table, W, b):
    x = x.astype(jnp.int32)
    rawsum = _sc_segment_sum(table, x)
    return _tc_finish(x, rawsum, table, W, b)


# speedup vs baseline: 2.5482x; 1.1141x over previous
import functools

import jax
import jax.numpy as jnp
from jax import lax
from jax.experimental import pallas as pl
from jax.experimental.pallas import tpu as pltpu
from jax.experimental.pallas import tpu_sc as plsc

_NC = 2
_NS = 16
_LANES = 16
_BB = 8
_VC = 2048


def _tc_retile(table_t):
    D, V = table_t.shape

    def body(t_ref, o_ref):
        blk = t_ref[...]
        eye = (jax.lax.broadcasted_iota(jnp.int32, (D, D), 0) ==
               jax.lax.broadcasted_iota(jnp.int32, (D, D), 1)
               ).astype(jnp.float32)
        rows = lax.dot_general(blk, eye, (((0,), (0,)), ((), ())),
                               preferred_element_type=jnp.float32,
                               precision=jax.lax.Precision.HIGHEST)

        @pl.when(pl.program_id(0) == 0)
        def _():
            o_ref[0:1, 0:D] = jnp.zeros((1, D), jnp.float32)

        @pl.when(pl.program_id(0) != 0)
        def _():
            o_ref[0:1, 0:D] = rows[0:1, :]

        o_ref[1:_VC, 0:D] = rows[1:_VC, :]

    return pl.pallas_call(
        body,
        out_shape=jax.ShapeDtypeStruct((V, 2 * D), jnp.float32),
        grid=((V + _VC - 1) // _VC,),
        in_specs=[pl.BlockSpec((D, _VC), lambda i: (0, i))],
        out_specs=pl.BlockSpec((_VC, 2 * D), lambda i: (i, 0)),
    )(table_t)


def _sc_segment_sum(table2, x):
    V2, D = table2.shape
    nblk_x, blk_len = x.shape
    B = nblk_x * _BB
    L = blk_len // _BB
    nw = _NC * _NS
    n_per = B // nw
    n_blocks = n_per // _BB
    n_chunks = D // _LANES
    idx_vecs = (n_blocks * blk_len) // _LANES
    mesh = plsc.VectorSubcoreMesh(core_axis_name="c", subcore_axis_name="s")

    @functools.partial(
        pl.kernel,
        out_type=jax.ShapeDtypeStruct((B, D), jnp.float32),
        mesh=mesh,
        compiler_params=pltpu.CompilerParams(use_tc_tiling_on_sc=False),
        scratch_types=[
            pltpu.VMEM((n_blocks, blk_len), jnp.int32),
            pltpu.VMEM((2, blk_len, D), jnp.float32),
            pltpu.VMEM((2, _BB, D), jnp.float32),
            pltpu.SemaphoreType.DMA,
            pltpu.SemaphoreType.DMA,
        ],
    )
    def k(table_hbm, x_hbm, out_hbm, idx_v, rows_v, acc_v, sem0, sem1):
        wid = lax.axis_index("s") * _NC + lax.axis_index("c")
        base = wid * n_per
        sems = (sem0, sem1)

        pltpu.sync_copy(x_hbm.at[pl.ds(wid * n_blocks, n_blocks)], idx_v)

        @plsc.parallel_loop(0, n_blocks)
        def _(r):
            for v in range(blk_len // _LANES):
                sl = pl.ds(v * _LANES, _LANES)
                w = idx_v[r, sl]
                idx_v[r, sl] = w + w

        def gather(slot, blk):
            return pltpu.make_async_copy(
                table_hbm.at[idx_v.at[blk]],
                rows_v.at[slot], sems[slot])

        def compute(slot, blk):
            @pl.loop(0, _BB)
            def _(i):
                init = tuple(rows_v[slot, i * L, pl.ds(c * _LANES, _LANES)]
                             for c in range(n_chunks))

                @plsc.parallel_loop(1, L, unroll=7, carry=init)
                def accs(r, carry):
                    return tuple(
                        carry[c] + rows_v[slot, i * L + r,
                                          pl.ds(c * _LANES, _LANES)]
                        for c in range(n_chunks))

                for c in range(n_chunks):
                    acc_v[slot, i, pl.ds(c * _LANES, _LANES)] = accs[c]

            pltpu.sync_copy(acc_v.at[slot],
                            out_hbm.at[pl.ds(base + blk * _BB, _BB)])

        gather(0, 0).start()

        @pl.loop(0, n_blocks // 2)
        def _(j):
            blk = j * 2
            gather(1, blk + 1).start()
            gather(0, blk).wait()
            compute(0, blk)

            @pl.when(blk + 2 < n_blocks)
            def _():
                gather(0, blk + 2).start()

            gather(1, blk + 1).wait()
            compute(1, blk + 1)

    return k(table2, x)


def _tc_finish(rawsum, W, b, L):
    B, D = rawsum.shape
    nout = W.shape[0]

    def body(s_ref, w_ref, b_ref, o_ref):
        pooled = s_ref[...] * (1.0 / L)
        o_ref[...] = lax.dot_general(
            pooled, w_ref[...], (((1,), (1,)), ((), ())),
            preferred_element_type=jnp.float32) + b_ref[...]

    tb = 2048
    return pl.pallas_call(
        body,
        out_shape=jax.ShapeDtypeStruct((B, nout), jnp.float32),
        grid=(B // tb,),
        in_specs=[
            pl.BlockSpec((tb, D), lambda i: (i, 0)),
            pl.BlockSpec((nout, D), lambda i: (0, 0)),
            pl.BlockSpec((1, nout), lambda i: (0, 0)),
        ],
        out_specs=pl.BlockSpec((tb, nout), lambda i: (i, 0)),
    )(rawsum, W, b.reshape(1, nout))


def kernel(x, table, W, b):
    B, L = x.shape
    V, D = table.shape
    x = x.astype(jnp.int32)
    conv = _tc_retile(table.T)
    table2 = conv.reshape(2 * V, D)
    rawsum = _sc_segment_sum(table2, x.reshape(B // _BB, _BB * L))
    return _tc_finish(rawsum, W, b, L)


# speedup vs baseline: 2.8928x; 1.1352x over previous
import functools

import jax
import jax.numpy as jnp
from jax import lax
from jax.experimental import pallas as pl
from jax.experimental.pallas import tpu as pltpu
from jax.experimental.pallas import tpu_sc as plsc

_NC = 2
_NS = 16
_LANES = 16
_BB = 8
_VC = 2048


def _tc_retile(table_t):
    D, V = table_t.shape

    nblk = (V // 2 + _VC - 1) // _VC

    def body(lo_ref, hi_ref, o_ref):
        eye = (jax.lax.broadcasted_iota(jnp.int32, (D, D), 0) ==
               jax.lax.broadcasted_iota(jnp.int32, (D, D), 1)
               ).astype(jnp.float32)

        def tr(blk):
            return lax.dot_general(blk, eye, (((0,), (0,)), ((), ())),
                                   preferred_element_type=jnp.float32,
                                   precision=jax.lax.Precision.HIGHEST)

        o_ref[:, 0:D] = tr(lo_ref[...])
        o_ref[:, D:2 * D] = tr(hi_ref[...])

        @pl.when(pl.program_id(0) == 0)
        def _():
            o_ref[0:1, 0:D] = jnp.zeros((1, D), jnp.float32)

    return pl.pallas_call(
        body,
        out_shape=jax.ShapeDtypeStruct((nblk * _VC, 2 * D), jnp.float32),
        grid=(nblk,),
        in_specs=[
            pl.BlockSpec((D, _VC), lambda i: (0, i)),
            pl.BlockSpec(
                (D, _VC),
                lambda i: (0, jnp.minimum(i + nblk, (V - 1) // _VC))),
        ],
        out_specs=pl.BlockSpec((_VC, 2 * D), lambda i: (i, 0)),
    )(table_t, table_t)


def _sc_segment_sum(table2, x, half):
    V2, D = table2.shape
    nblk_x, blk_len = x.shape
    B = nblk_x * _BB
    L = blk_len // _BB
    nw = _NC * _NS
    n_per = B // nw
    n_blocks = n_per // _BB
    n_chunks = D // _LANES
    idx_vecs = (n_blocks * blk_len) // _LANES
    mesh = plsc.VectorSubcoreMesh(core_axis_name="c", subcore_axis_name="s")

    @functools.partial(
        pl.kernel,
        out_type=jax.ShapeDtypeStruct((B, D), jnp.float32),
        mesh=mesh,
        compiler_params=pltpu.CompilerParams(use_tc_tiling_on_sc=False),
        scratch_types=[
            pltpu.VMEM((n_blocks, blk_len), jnp.int32),
            pltpu.VMEM((2, blk_len, D), jnp.float32),
            pltpu.VMEM((2, _BB, D), jnp.float32),
            pltpu.SemaphoreType.DMA,
            pltpu.SemaphoreType.DMA,
        ],
    )
    def k(table_hbm, x_hbm, out_hbm, idx_v, rows_v, acc_v, sem0, sem1):
        wid = lax.axis_index("s") * _NC + lax.axis_index("c")
        base = wid * n_per
        sems = (sem0, sem1)

        pltpu.sync_copy(x_hbm.at[pl.ds(wid * n_blocks, n_blocks)], idx_v)

        @plsc.parallel_loop(0, n_blocks)
        def _(r):
            for v in range(blk_len // _LANES):
                sl = pl.ds(v * _LANES, _LANES)
                w = idx_v[r, sl]
                w2 = w + w
                idx_v[r, sl] = jnp.where(w < half, w2, w2 - (2 * half - 1))

        def gather(slot, blk):
            return pltpu.make_async_copy(
                table_hbm.at[idx_v.at[blk]],
                rows_v.at[slot], sems[slot])

        def compute(slot, blk):
            @pl.loop(0, _BB)
            def _(i):
                init = tuple(rows_v[slot, i * L, pl.ds(c * _LANES, _LANES)]
                             for c in range(n_chunks))

                @plsc.parallel_loop(1, L, unroll=7, carry=init)
                def accs(r, carry):
                    return tuple(
                        carry[c] + rows_v[slot, i * L + r,
                                          pl.ds(c * _LANES, _LANES)]
                        for c in range(n_chunks))

                for c in range(n_chunks):
                    acc_v[slot, i, pl.ds(c * _LANES, _LANES)] = accs[c]

            pltpu.sync_copy(acc_v.at[slot],
                            out_hbm.at[pl.ds(base + blk * _BB, _BB)])

        gather(0, 0).start()

        @pl.loop(0, n_blocks // 2)
        def _(j):
            blk = j * 2
            gather(1, blk + 1).start()
            gather(0, blk).wait()
            compute(0, blk)

            @pl.when(blk + 2 < n_blocks)
            def _():
                gather(0, blk + 2).start()

            gather(1, blk + 1).wait()
            compute(1, blk + 1)

    return k(table2, x)


def _tc_finish(rawsum, W, b, L):
    B, D = rawsum.shape
    nout = W.shape[0]

    def body(s_ref, w_ref, b_ref, o_ref):
        pooled = s_ref[...] * (1.0 / L)
        o_ref[...] = lax.dot_general(
            pooled, w_ref[...], (((1,), (1,)), ((), ())),
            preferred_element_type=jnp.float32) + b_ref[...]

    tb = 2048
    return pl.pallas_call(
        body,
        out_shape=jax.ShapeDtypeStruct((B, nout), jnp.float32),
        grid=(B // tb,),
        in_specs=[
            pl.BlockSpec((tb, D), lambda i: (i, 0)),
            pl.BlockSpec((nout, D), lambda i: (0, 0)),
            pl.BlockSpec((1, nout), lambda i: (0, 0)),
        ],
        out_specs=pl.BlockSpec((tb, nout), lambda i: (i, 0)),
    )(rawsum, W, b.reshape(1, nout))


def kernel(x, table, W, b):
    B, L = x.shape
    V, D = table.shape
    x = x.astype(jnp.int32)
    conv = _tc_retile(table.T)
    half = conv.shape[0]
    table2 = conv.reshape(2 * half, D)
    rawsum = _sc_segment_sum(table2, x.reshape(B // _BB, _BB * L), half)
    return _tc_finish(rawsum, W, b, L)


# speedup vs baseline: 4.1460x; 1.4332x over previous
import functools

import jax
import jax.numpy as jnp
from jax import lax
from jax.experimental import pallas as pl
from jax.experimental.pallas import tpu as pltpu
from jax.experimental.pallas import tpu_sc as plsc

_NC = 2
_NS = 16
_LANES = 16
_BB = 8
_VC = 2048


def _tc_retile(table_t):
    D, V = table_t.shape

    nblk = (V // 2 + _VC - 1) // _VC

    def body(lo_ref, hi_ref, o_ref):
        o_ref[:, 0:D] = lo_ref[...].T
        o_ref[:, D:2 * D] = hi_ref[...].T

        @pl.when(pl.program_id(0) == 0)
        def _():
            o_ref[0:1, 0:D] = jnp.zeros((1, D), jnp.float32)

    return pl.pallas_call(
        body,
        out_shape=jax.ShapeDtypeStruct((nblk * _VC, 2 * D), jnp.float32),
        grid=(nblk,),
        in_specs=[
            pl.BlockSpec((D, _VC), lambda i: (0, i)),
            pl.BlockSpec(
                (D, _VC),
                lambda i: (0, jnp.minimum(i + nblk, (V - 1) // _VC))),
        ],
        out_specs=pl.BlockSpec((_VC, 2 * D), lambda i: (i, 0)),
    )(table_t, table_t)


def _sc_segment_sum(table2, x, half):
    V2, D = table2.shape
    nblk_x, blk_len = x.shape
    B = nblk_x * _BB
    L = blk_len // _BB
    nw = _NC * _NS
    n_per = B // nw
    n_blocks = n_per // _BB
    n_chunks = D // _LANES
    idx_vecs = (n_blocks * blk_len) // _LANES
    mesh = plsc.VectorSubcoreMesh(core_axis_name="c", subcore_axis_name="s")

    @functools.partial(
        pl.kernel,
        out_type=jax.ShapeDtypeStruct((B, D), jnp.float32),
        mesh=mesh,
        compiler_params=pltpu.CompilerParams(use_tc_tiling_on_sc=False),
        scratch_types=[
            pltpu.VMEM((n_blocks, blk_len), jnp.int32),
            pltpu.VMEM((2, blk_len, D), jnp.float32),
            pltpu.VMEM((2, _BB, D), jnp.float32),
            pltpu.SemaphoreType.DMA,
            pltpu.SemaphoreType.DMA,
        ],
    )
    def k(table_hbm, x_hbm, out_hbm, idx_v, rows_v, acc_v, sem0, sem1):
        wid = lax.axis_index("s") * _NC + lax.axis_index("c")
        base = wid * n_per
        sems = (sem0, sem1)

        pltpu.sync_copy(x_hbm.at[pl.ds(wid * n_blocks, n_blocks)], idx_v)

        @plsc.parallel_loop(0, n_blocks)
        def _(r):
            for v in range(blk_len // _LANES):
                sl = pl.ds(v * _LANES, _LANES)
                w = idx_v[r, sl]
                w2 = w + w
                idx_v[r, sl] = jnp.where(w < half, w2, w2 - (2 * half - 1))

        def gather(slot, blk):
            return pltpu.make_async_copy(
                table_hbm.at[idx_v.at[blk]],
                rows_v.at[slot], sems[slot])

        def compute(slot, blk):
            @pl.loop(0, _BB)
            def _(i):
                init = tuple(rows_v[slot, i * L, pl.ds(c * _LANES, _LANES)]
                             for c in range(n_chunks))

                @plsc.parallel_loop(1, L, unroll=7, carry=init)
                def accs(r, carry):
                    return tuple(
                        carry[c] + rows_v[slot, i * L + r,
                                          pl.ds(c * _LANES, _LANES)]
                        for c in range(n_chunks))

                for c in range(n_chunks):
                    acc_v[slot, i, pl.ds(c * _LANES, _LANES)] = accs[c]

            pltpu.sync_copy(acc_v.at[slot],
                            out_hbm.at[pl.ds(base + blk * _BB, _BB)])

        gather(0, 0).start()

        @pl.loop(0, n_blocks // 2)
        def _(j):
            blk = j * 2
            gather(1, blk + 1).start()
            gather(0, blk).wait()
            compute(0, blk)

            @pl.when(blk + 2 < n_blocks)
            def _():
                gather(0, blk + 2).start()

            gather(1, blk + 1).wait()
            compute(1, blk + 1)

    return k(table2, x)


def _tc_finish(rawsum, W, b, L):
    B, D = rawsum.shape
    nout = W.shape[0]

    def body(s_ref, w_ref, b_ref, o_ref):
        pooled = s_ref[...] * (1.0 / L)
        o_ref[...] = lax.dot_general(
            pooled, w_ref[...], (((1,), (1,)), ((), ())),
            preferred_element_type=jnp.float32) + b_ref[...]

    tb = 2048
    return pl.pallas_call(
        body,
        out_shape=jax.ShapeDtypeStruct((B, nout), jnp.float32),
        grid=(B // tb,),
        in_specs=[
            pl.BlockSpec((tb, D), lambda i: (i, 0)),
            pl.BlockSpec((nout, D), lambda i: (0, 0)),
            pl.BlockSpec((1, nout), lambda i: (0, 0)),
        ],
        out_specs=pl.BlockSpec((tb, nout), lambda i: (i, 0)),
    )(rawsum, W, b.reshape(1, nout))


def kernel(x, table, W, b):
    B, L = x.shape
    V, D = table.shape
    x = x.astype(jnp.int32)
    conv = _tc_retile(table.T)
    half = conv.shape[0]
    table2 = conv.reshape(2 * half, D)
    rawsum = _sc_segment_sum(table2, x.reshape(B // _BB, _BB * L), half)
    return _tc_finish(rawsum, W, b, L)


# speedup vs baseline: 4.8070x; 1.1594x over previous
import functools

import jax
import jax.numpy as jnp
from jax import lax
from jax.experimental import pallas as pl
from jax.experimental.pallas import tpu as pltpu
from jax.experimental.pallas import tpu_sc as plsc

_NC = 2
_NS = 16
_LANES = 16
_BB = 8
_VC = 4096


def _tc_retile(table_t):
    D, V = table_t.shape

    nblk = (V // 2 + _VC - 1) // _VC

    def body(lo_ref, hi_ref, o_ref):
        o_ref[:, 0:D] = lo_ref[...].T
        o_ref[:, D:2 * D] = hi_ref[...].T

        @pl.when(pl.program_id(0) == 0)
        def _():
            o_ref[0:1, 0:D] = jnp.zeros((1, D), jnp.float32)

    return pl.pallas_call(
        body,
        out_shape=jax.ShapeDtypeStruct((nblk * _VC, 2 * D), jnp.float32),
        grid=(nblk,),
        in_specs=[
            pl.BlockSpec((D, _VC), lambda i: (0, i)),
            pl.BlockSpec(
                (D, _VC),
                lambda i: (0, jnp.minimum(i + nblk, (V - 1) // _VC))),
        ],
        out_specs=pl.BlockSpec((_VC, 2 * D), lambda i: (i, 0)),
    )(table_t, table_t)


def _sc_segment_sum(table2, x, half):
    V2, D = table2.shape
    nblk_x, blk_len = x.shape
    B = nblk_x * _BB
    L = blk_len // _BB
    nw = _NC * _NS
    n_per = B // nw
    n_blocks = n_per // _BB
    n_chunks = D // _LANES
    idx_vecs = (n_blocks * blk_len) // _LANES
    mesh = plsc.VectorSubcoreMesh(core_axis_name="c", subcore_axis_name="s")

    @functools.partial(
        pl.kernel,
        out_type=jax.ShapeDtypeStruct((B, D), jnp.float32),
        mesh=mesh,
        compiler_params=pltpu.CompilerParams(use_tc_tiling_on_sc=False),
        scratch_types=[
            pltpu.VMEM((n_blocks, blk_len), jnp.int32),
            pltpu.VMEM((2, blk_len, D), jnp.float32),
            pltpu.VMEM((2, _BB, D), jnp.float32),
            pltpu.SemaphoreType.DMA,
            pltpu.SemaphoreType.DMA,
        ],
    )
    def k(table_hbm, x_hbm, out_hbm, idx_v, rows_v, acc_v, sem0, sem1):
        wid = lax.axis_index("s") * _NC + lax.axis_index("c")
        base = wid * n_per
        sems = (sem0, sem1)

        pltpu.sync_copy(x_hbm.at[pl.ds(wid * n_blocks, n_blocks)], idx_v)

        @plsc.parallel_loop(0, n_blocks)
        def _(r):
            for v in range(blk_len // _LANES):
                sl = pl.ds(v * _LANES, _LANES)
                w = idx_v[r, sl]
                w2 = w + w
                idx_v[r, sl] = jnp.where(w < half, w2, w2 - (2 * half - 1))

        def gather(slot, blk):
            return pltpu.make_async_copy(
                table_hbm.at[idx_v.at[blk]],
                rows_v.at[slot], sems[slot])

        def compute(slot, blk):
            @pl.loop(0, _BB)
            def _(i):
                init = tuple(rows_v[slot, i * L, pl.ds(c * _LANES, _LANES)]
                             for c in range(n_chunks))

                @plsc.parallel_loop(1, L, unroll=7, carry=init)
                def accs(r, carry):
                    return tuple(
                        carry[c] + rows_v[slot, i * L + r,
                                          pl.ds(c * _LANES, _LANES)]
                        for c in range(n_chunks))

                for c in range(n_chunks):
                    acc_v[slot, i, pl.ds(c * _LANES, _LANES)] = accs[c]

            pltpu.sync_copy(acc_v.at[slot],
                            out_hbm.at[pl.ds(base + blk * _BB, _BB)])

        gather(0, 0).start()

        @pl.loop(0, n_blocks // 2)
        def _(j):
            blk = j * 2
            gather(1, blk + 1).start()
            gather(0, blk).wait()
            compute(0, blk)

            @pl.when(blk + 2 < n_blocks)
            def _():
                gather(0, blk + 2).start()

            gather(1, blk + 1).wait()
            compute(1, blk + 1)

    return k(table2, x)


def _tc_finish(rawsum, W, b, L):
    B, D = rawsum.shape
    nout = W.shape[0]

    def body(s_ref, w_ref, b_ref, o_ref):
        pooled = s_ref[...] * (1.0 / L)
        o_ref[...] = lax.dot_general(
            pooled, w_ref[...], (((1,), (1,)), ((), ())),
            preferred_element_type=jnp.float32) + b_ref[...]

    tb = 2048
    return pl.pallas_call(
        body,
        out_shape=jax.ShapeDtypeStruct((B, nout), jnp.float32),
        grid=(B // tb,),
        in_specs=[
            pl.BlockSpec((tb, D), lambda i: (i, 0)),
            pl.BlockSpec((nout, D), lambda i: (0, 0)),
            pl.BlockSpec((1, nout), lambda i: (0, 0)),
        ],
        out_specs=pl.BlockSpec((tb, nout), lambda i: (i, 0)),
    )(rawsum, W, b.reshape(1, nout))


def kernel(x, table, W, b):
    B, L = x.shape
    V, D = table.shape
    x = x.astype(jnp.int32)
    conv = _tc_retile(table.T)
    half = conv.shape[0]
    table2 = conv.reshape(2 * half, D)
    rawsum = _sc_segment_sum(table2, x.reshape(B // _BB, _BB * L), half)
    return _tc_finish(rawsum, W, b, L)


# speedup vs baseline: 5.0696x; 1.0546x over previous
import functools

import jax
import jax.numpy as jnp
from jax import lax
from jax.experimental import pallas as pl
from jax.experimental.pallas import tpu as pltpu
from jax.experimental.pallas import tpu_sc as plsc

_NC = 2
_NS = 16
_LANES = 16
_BB = 8
_VC = 6144


def _tc_retile(table_t):
    D, V = table_t.shape

    nblk = (V // 2 + _VC - 1) // _VC

    def body(lo_ref, hi_ref, o_ref):
        o_ref[:, 0:D] = lo_ref[...].T
        o_ref[:, D:2 * D] = hi_ref[...].T

        @pl.when(pl.program_id(0) == 0)
        def _():
            o_ref[0:1, 0:D] = jnp.zeros((1, D), jnp.float32)

    return pl.pallas_call(
        body,
        out_shape=jax.ShapeDtypeStruct((nblk * _VC, 2 * D), jnp.float32),
        grid=(nblk,),
        in_specs=[
            pl.BlockSpec((D, _VC), lambda i: (0, i)),
            pl.BlockSpec(
                (D, _VC),
                lambda i: (0, jnp.minimum(i + nblk, (V - 1) // _VC))),
        ],
        out_specs=pl.BlockSpec((_VC, 2 * D), lambda i: (i, 0)),
    )(table_t, table_t)


def _sc_segment_sum(table2, x, half):
    V2, D = table2.shape
    nblk_x, blk_len = x.shape
    B = nblk_x * _BB
    L = blk_len // _BB
    nw = _NC * _NS
    n_per = B // nw
    n_blocks = n_per // _BB
    n_chunks = D // _LANES
    idx_vecs = (n_blocks * blk_len) // _LANES
    mesh = plsc.VectorSubcoreMesh(core_axis_name="c", subcore_axis_name="s")

    @functools.partial(
        pl.kernel,
        out_type=jax.ShapeDtypeStruct((B, D), jnp.float32),
        mesh=mesh,
        compiler_params=pltpu.CompilerParams(use_tc_tiling_on_sc=False),
        scratch_types=[
            pltpu.VMEM((n_blocks, blk_len), jnp.int32),
            pltpu.VMEM((2, blk_len, D), jnp.float32),
            pltpu.VMEM((2, _BB, D), jnp.float32),
            pltpu.SemaphoreType.DMA,
            pltpu.SemaphoreType.DMA,
        ],
    )
    def k(table_hbm, x_hbm, out_hbm, idx_v, rows_v, acc_v, sem0, sem1):
        wid = lax.axis_index("s") * _NC + lax.axis_index("c")
        base = wid * n_per
        sems = (sem0, sem1)

        pltpu.sync_copy(x_hbm.at[pl.ds(wid * n_blocks, n_blocks)], idx_v)

        @plsc.parallel_loop(0, n_blocks)
        def _(r):
            for v in range(blk_len // _LANES):
                sl = pl.ds(v * _LANES, _LANES)
                w = idx_v[r, sl]
                w2 = w + w
                idx_v[r, sl] = jnp.where(w < half, w2, w2 - (2 * half - 1))

        def gather(slot, blk):
            return pltpu.make_async_copy(
                table_hbm.at[idx_v.at[blk]],
                rows_v.at[slot], sems[slot])

        def compute(slot, blk):
            @pl.loop(0, _BB)
            def _(i):
                init = tuple(rows_v[slot, i * L, pl.ds(c * _LANES, _LANES)]
                             for c in range(n_chunks))

                @plsc.parallel_loop(1, L, unroll=7, carry=init)
                def accs(r, carry):
                    return tuple(
                        carry[c] + rows_v[slot, i * L + r,
                                          pl.ds(c * _LANES, _LANES)]
                        for c in range(n_chunks))

                for c in range(n_chunks):
                    acc_v[slot, i, pl.ds(c * _LANES, _LANES)] = accs[c]

            pltpu.sync_copy(acc_v.at[slot],
                            out_hbm.at[pl.ds(base + blk * _BB, _BB)])

        gather(0, 0).start()

        @pl.loop(0, n_blocks // 2)
        def _(j):
            blk = j * 2
            gather(1, blk + 1).start()
            gather(0, blk).wait()
            compute(0, blk)

            @pl.when(blk + 2 < n_blocks)
            def _():
                gather(0, blk + 2).start()

            gather(1, blk + 1).wait()
            compute(1, blk + 1)

    return k(table2, x)


def _tc_finish(rawsum, W, b, L):
    B, D = rawsum.shape
    nout = W.shape[0]

    def body(s_ref, w_ref, b_ref, o_ref):
        pooled = s_ref[...] * (1.0 / L)
        o_ref[...] = lax.dot_general(
            pooled, w_ref[...], (((1,), (1,)), ((), ())),
            preferred_element_type=jnp.float32) + b_ref[...]

    tb = 2048
    return pl.pallas_call(
        body,
        out_shape=jax.ShapeDtypeStruct((B, nout), jnp.float32),
        grid=(B // tb,),
        in_specs=[
            pl.BlockSpec((tb, D), lambda i: (i, 0)),
            pl.BlockSpec((nout, D), lambda i: (0, 0)),
            pl.BlockSpec((1, nout), lambda i: (0, 0)),
        ],
        out_specs=pl.BlockSpec((tb, nout), lambda i: (i, 0)),
    )(rawsum, W, b.reshape(1, nout))


def kernel(x, table, W, b):
    B, L = x.shape
    V, D = table.shape
    x = x.astype(jnp.int32)
    conv = _tc_retile(table.T)
    half = conv.shape[0]
    table2 = conv.reshape(2 * half, D)
    rawsum = _sc_segment_sum(table2, x.reshape(B // _BB, _BB * L), half)
    return _tc_finish(rawsum, W, b, L)
